# Initial kernel scaffold; baseline (speedup 1.0000x reference)
import jax
import jax.numpy as jnp
from jax.experimental import pallas as pl


def kernel(image1, image2, pair_offsets, pair_thresholds):
    raise NotImplementedError("write your pallas kernel here")


# speedup vs baseline: 7.0433x; 7.0433x over previous
import functools
import numpy as np
import jax
import jax.numpy as jnp
from jax import lax
from jax.experimental import pallas as pl

B, H, W = 2, 512, 512
MAX_KPTS = 1024
NUM_SCALES = 3
DIFF_ITERS = 3
KAPPA = 0.05
THRESHOLD = 0.001
AKAZE_NMS = 5
ORI_PATCH = 15
ORI_SIGMA = 2.5
NUM_PAIRS = 256
SINK_ITERS = 20
EPSILON = 1.0
UNUSED = 1.0
NMS_RADIUS = 3
SCORE_THRESH = 0.0
MAX_RADIUS = 16
BORDER = MAX_RADIUS

NEG = -1e30
NPAD = 1152


def _shift1e(x, d, axis):
    if axis == 0:
        if d == 1:
            return jnp.concatenate([x[1:, :], x[-1:, :]], axis=0)
        return jnp.concatenate([x[:1, :], x[:-1, :]], axis=0)
    if d == 1:
        return jnp.concatenate([x[:, 1:], x[:, -1:]], axis=1)
    return jnp.concatenate([x[:, :1], x[:, :-1]], axis=1)


def _shift(x, dy, dx):
    if dy != 0:
        x = _shift1e(x, dy, 0)
    if dx != 0:
        x = _shift1e(x, dx, 1)
    return x


def _maxpool(x, radius):
    up = x
    dn = x
    m = x
    for _ in range(radius):
        up = _shift1e(up, 1, 0)
        dn = _shift1e(dn, -1, 0)
        m = jnp.maximum(m, jnp.maximum(up, dn))
    up = m
    dn = m
    out = m
    for _ in range(radius):
        up = _shift1e(up, 1, 1)
        dn = _shift1e(dn, -1, 1)
        out = jnp.maximum(out, jnp.maximum(up, dn))
    return out


def _diffusion_step(L):
    Le = _shift(L, 0, 1)
    Lw = _shift(L, 0, -1)
    Ls = _shift(L, 1, 0)
    Ln = _shift(L, -1, 0)
    Lx = 0.5 * (Le - Lw)
    Ly = 0.5 * (Ls - Ln)
    g = jnp.exp(-((Lx * Lx + Ly * Ly) / (KAPPA * KAPPA)))
    fe = Le - L
    fw = L - Lw
    fs = Ls - L
    fn = L - Ln
    ge = 0.5 * (g + _shift(g, 0, 1))
    gw = 0.5 * (g + _shift(g, 0, -1))
    gs = 0.5 * (g + _shift(g, 1, 0))
    gn = 0.5 * (g + _shift(g, -1, 0))
    return L + 0.25 * (ge * fe - gw * fw + gs * fs - gn * fn)


def _hessian_response(L):
    Le = _shift(L, 0, 1)
    Lw = _shift(L, 0, -1)
    Ls = _shift(L, 1, 0)
    Ln = _shift(L, -1, 0)
    Lxx = Le - 2.0 * L + Lw
    Lyy = Ls - 2.0 * L + Ln
    Lxy = 0.25 * (_shift(L, 1, 1) - _shift(L, 1, -1)
                  - _shift(L, -1, 1) + _shift(L, -1, -1))
    return Lxx * Lyy - Lxy * Lxy


def _shiftz(x, d, axis):
    if d == 0:
        return x
    n = abs(d)
    if axis == 0:
        z = jnp.zeros((n, x.shape[1]), x.dtype)
        if d > 0:
            return jnp.concatenate([x[n:, :], z], axis=0)
        return jnp.concatenate([z, x[:-n, :]], axis=0)
    z = jnp.zeros((x.shape[0], n), x.dtype)
    if d > 0:
        return jnp.concatenate([x[:, n:], z], axis=1)
    return jnp.concatenate([z, x[:, :-n]], axis=1)


_rr = ORI_PATCH // 2
_ax = np.arange(-_rr, _rr + 1, dtype=np.float64)
_g1 = np.exp(-(_ax * _ax) / (2.0 * ORI_SIGMA * ORI_SIGMA)).astype(np.float32)
_W1D = (_g1 / _g1.sum()).tolist()


def _gauss_smooth(x):
    for axis in (1, 0):
        acc = _W1D[_rr] * x
        for t in range(1, _rr + 1):
            acc = acc + _W1D[_rr + t] * (_shiftz(x, t, axis)
                                         + _shiftz(x, -t, axis))
        x = acc
    return x


def _akaze_body(x_ref, ms_ref, c_ref, s_ref):
    L = x_ref[0]
    scores = jnp.zeros_like(L)
    for _s in range(NUM_SCALES):
        L = lax.fori_loop(0, DIFF_ITERS, lambda i, Lc: _diffusion_step(Lc), L)
        r = _hessian_response(L)
        keep = (r >= _maxpool(r, AKAZE_NMS // 2)) & (r > THRESHOLD)
        scores = jnp.maximum(scores, jnp.where(keep, r, 0.0))

    Lx = 0.5 * (_shift(L, 0, 1) - _shift(L, 0, -1))
    Ly = 0.5 * (_shift(L, 1, 0) - _shift(L, -1, 0))
    sx = _gauss_smooth(Lx)
    sy = _gauss_smooth(Ly)
    rn = jnp.sqrt(sx * sx + sy * sy)
    safe = rn > 0.0
    rs = jnp.maximum(rn, 1e-30)
    c_ref[0] = jnp.where(safe, sx / rs, 1.0)
    s_ref[0] = jnp.where(safe, sy / rs, 0.0)

    nms = scores >= _maxpool(scores, NMS_RADIUS)
    yy = lax.broadcasted_iota(jnp.int32, (H, W), 0)
    xx = lax.broadcasted_iota(jnp.int32, (H, W), 1)
    bm = ((yy >= BORDER) & (yy < H - BORDER)
          & (xx >= BORDER) & (xx < W - BORDER))
    valid = nms & (scores > SCORE_THRESH) & bm
    ms_ref[0] = jnp.where(valid, scores, -jnp.inf)


def _akaze_all(x):
    n = x.shape[0]
    out = jax.ShapeDtypeStruct((n, H, W), jnp.float32)
    return pl.pallas_call(
        _akaze_body,
        grid=(n,),
        in_specs=[pl.BlockSpec((1, H, W), lambda i: (i, 0, 0))],
        out_specs=[pl.BlockSpec((1, H, W), lambda i: (i, 0, 0))] * 3,
        out_shape=[out, out, out],
    )(x)


def _sink_body(d1_ref, d2_ref, out_ref):
    d1 = d1_ref[0]
    d2 = d2_ref[0]
    G = lax.dot_general(d1, d2, (((1,), (1,)), ((), ())),
                        preferred_element_type=jnp.float32)
    sq1 = jnp.sum(d1 * d1, axis=1, keepdims=True)
    sq2 = jnp.sum(d2 * d2, axis=1, keepdims=True)
    d2m = sq1 + jnp.transpose(sq2) - 2.0 * G
    sc = -jnp.sqrt(jnp.clip(d2m, 0.0, None) + 1e-12)

    scp = jnp.concatenate(
        [jnp.concatenate([sc, jnp.zeros((MAX_KPTS, NPAD - MAX_KPTS),
                                        jnp.float32)], axis=1),
         jnp.zeros((NPAD - MAX_KPTS, NPAD), jnp.float32)], axis=0)
    ri = lax.broadcasted_iota(jnp.int32, (NPAD, NPAD), 0)
    ci = lax.broadcasted_iota(jnp.int32, (NPAD, NPAD), 1)
    main = (ri < MAX_KPTS) & (ci < MAX_KPTS)
    bins = (((ri == MAX_KPTS) & (ci <= MAX_KPTS))
            | ((ci == MAX_KPTS) & (ri <= MAX_KPTS)))
    Z = jnp.where(main, scp, jnp.where(bins, UNUSED, NEG)) / EPSILON

    norm = -np.log(2.0 * MAX_KPTS)
    li = lax.broadcasted_iota(jnp.int32, (NPAD, 1), 0)
    log_mu = jnp.where(li < MAX_KPTS, norm,
                       jnp.where(li == MAX_KPTS,
                                 np.log(float(MAX_KPTS)) + norm, NEG))
    log_nu = jnp.transpose(log_mu)

    def body(_i, uv):
        u, v = uv
        t = Z + v
        m = jnp.max(t, axis=1, keepdims=True)
        u = log_mu - (m + jnp.log(jnp.sum(jnp.exp(t - m), axis=1,
                                          keepdims=True)))
        t = Z + u
        m = jnp.max(t, axis=0, keepdims=True)
        v = log_nu - (m + jnp.log(jnp.sum(jnp.exp(t - m), axis=0,
                                          keepdims=True)))
        return (u, v)

    u, v = lax.fori_loop(0, SINK_ITERS, body,
                         (jnp.zeros((NPAD, 1), jnp.float32),
                          jnp.zeros((1, NPAD), jnp.float32)))
    out_ref[0] = jnp.exp(Z + u + v - norm)


def _sink_all(d1, d2):
    return pl.pallas_call(
        _sink_body,
        grid=(B,),
        in_specs=[pl.BlockSpec((1, MAX_KPTS, NUM_PAIRS), lambda i: (i, 0, 0)),
                  pl.BlockSpec((1, MAX_KPTS, NUM_PAIRS), lambda i: (i, 0, 0))],
        out_specs=pl.BlockSpec((1, NPAD, NPAD), lambda i: (i, 0, 0)),
        out_shape=jax.ShapeDtypeStruct((B, NPAD, NPAD), jnp.float32),
    )(d1, d2)


def _gather_maps(img, yi, xi):
    n = img.shape[0]
    flat = img.reshape(n, -1)
    idx = (yi * W + xi).reshape(n, -1)
    out = jnp.take_along_axis(flat, idx, axis=1)
    return out.reshape(yi.shape)


def _descriptors(imgs, kpts, cmap, smap, offsets, thresholds):
    ky = kpts[..., 0]
    kx = kpts[..., 1]
    yi = jnp.clip(jnp.round(ky), 0, H - 1).astype(jnp.int32)
    xi = jnp.clip(jnp.round(kx), 0, W - 1).astype(jnp.int32)
    c = _gather_maps(cmap, yi, xi)[..., None]
    s = _gather_maps(smap, yi, xi)[..., None]
    ox1, oy1, ox2, oy2 = (offsets[:, 0], offsets[:, 1],
                          offsets[:, 2], offsets[:, 3])
    rx1 = c * ox1 - s * oy1
    ry1 = s * ox1 + c * oy1
    rx2 = c * ox2 - s * oy2
    ry2 = s * ox2 + c * oy2
    y1 = jnp.clip(jnp.round(ky[..., None] + ry1), 0, H - 1).astype(jnp.int32)
    x1 = jnp.clip(jnp.round(kx[..., None] + rx1), 0, W - 1).astype(jnp.int32)
    y2 = jnp.clip(jnp.round(ky[..., None] + ry2), 0, H - 1).astype(jnp.int32)
    x2 = jnp.clip(jnp.round(kx[..., None] + rx2), 0, W - 1).astype(jnp.int32)
    v1 = _gather_maps(imgs, y1, x1)
    v2 = _gather_maps(imgs, y2, x2)
    desc = v1 - v2 - thresholds
    return desc / (jnp.linalg.norm(desc, axis=-1, keepdims=True) + 1e-8)


def kernel(image1, image2, pair_offsets, pair_thresholds):
    X = jnp.concatenate([image1[:, 0], image2[:, 0]], axis=0)
    ms, cmap, smap = _akaze_all(X)

    vals, idx = lax.top_k(ms.reshape(2 * B, -1), MAX_KPTS)
    ys = idx // W
    xs = idx % W
    ok = jnp.isfinite(vals)
    kpts = jnp.where(ok[..., None], jnp.stack([ys, xs], -1), -1)
    kpts = kpts.astype(jnp.float32)

    desc = _descriptors(X, kpts, cmap, smap, pair_offsets, pair_thresholds)
    probs_pad = _sink_all(desc[:B], desc[B:])
    probs = probs_pad[:, :MAX_KPTS + 1, :MAX_KPTS + 1]
    return kpts[:B], kpts[B:], probs
